# Initial kernel scaffold; baseline (speedup 1.0000x reference)
import jax
import jax.numpy as jnp
from jax.experimental import pallas as pl


def kernel(x, edge_index, edge_weight, W1, b1, W2, b2):
    raise NotImplementedError("write your pallas kernel here")


# speedup vs baseline: 3.0722x; 3.0722x over previous
import functools

import jax
import jax.numpy as jnp
from jax import lax
from jax.experimental import pallas as pl
from jax.experimental.pallas import tpu as pltpu
from jax.experimental.pallas import tpu_sc as plsc

N = 10000
E = 320000
NP = 10240
D_IN = 128
H = 256
HALF = 128

E_FULL = E + N
ROWS = 2688
E_PAD = ROWS * 128
EPT = E_PAD // 32
RC = ROWS // 16
RPT = NP // 16
NCHUNK = RPT // 128

_mesh = plsc.VectorSubcoreMesh(core_axis_name="c", subcore_axis_name="s")
_sc_params = pltpu.CompilerParams(needs_layout_passes=False)


@functools.partial(
    pl.kernel,
    out_type=jax.ShapeDtypeStruct((32 * NP,), jnp.float32),
    mesh=_mesh,
    scratch_types=[
        pltpu.VMEM((EPT,), jnp.int32),
        pltpu.VMEM((EPT,), jnp.float32),
        pltpu.VMEM((NP,), jnp.float32),
    ],
    compiler_params=_sc_params,
)
def _deg_kernel(dst_hbm, ew_hbm, parts_hbm, dst_v, ew_v, hist_v):
    cid = lax.axis_index("c")
    sid = lax.axis_index("s")
    wid = sid * 2 + cid
    base = wid * EPT
    pltpu.sync_copy(dst_hbm.at[pl.ds(base, EPT)], dst_v)
    pltpu.sync_copy(ew_hbm.at[pl.ds(base, EPT)], ew_v)

    zeros16 = jnp.zeros((16,), jnp.float32)

    def zbody(i, carry):
        hist_v[pl.ds(i * 16, 16)] = zeros16
        return carry

    lax.fori_loop(0, NP // 16, zbody, 0)

    def rbody(r, carry):
        idx = dst_v[pl.ds(r * 16, 16)]
        w = ew_v[pl.ds(r * 16, 16)]
        plsc.addupdate_scatter(hist_v, [idx], w)
        return carry

    lax.fori_loop(0, EPT // 16, rbody, 0)
    pltpu.sync_copy(hist_v, parts_hbm.at[pl.ds(wid * NP, NP)])


@functools.partial(
    pl.kernel,
    out_type=(
        jax.ShapeDtypeStruct((NP, HALF), jnp.float32),
        jax.ShapeDtypeStruct((NP, HALF), jnp.float32),
    ),
    mesh=_mesh,
    scratch_types=[
        pltpu.VMEM((8, 128), jnp.int32),
        pltpu.VMEM((8, 128), jnp.int32),
        pltpu.VMEM((8, 128), jnp.float32),
        pltpu.VMEM((128, HALF), jnp.float32),
        pltpu.VMEM((128, HALF), jnp.float32),
        pltpu.VMEM_SHARED((NP, HALF), jnp.float32),
        pltpu.SemaphoreType.DMA,
    ],
    compiler_params=_sc_params,
)
def _agg_kernel(src_hbm, dst_hbm, ew_hbm, g0_hbm, g1_hbm,
                out0_hbm, out1_hbm,
                src_v, dst_v, ew_v, rows_v, msg_v, acc, gsem):
    cid = lax.axis_index("c")
    sid = lax.axis_index("s")

    zeros16 = jnp.zeros((16,), jnp.float32)

    def zbody(i, carry):
        for k in range(HALF // 16):
            msg_v[i, pl.ds(k * 16, 16)] = zeros16
        return carry

    lax.fori_loop(0, 128, zbody, 0)
    for i in range(NCHUNK):
        pltpu.sync_copy(msg_v, acc.at[pl.ds(sid * RPT + i * 128, 128)])
    plsc.subcore_barrier()

    def run(g_hbm, out_hbm):
        def gbody(gi, carry):
            base = sid * RC + gi * 8
            pltpu.sync_copy(src_hbm.at[pl.ds(base, 8)], src_v)
            pltpu.sync_copy(dst_hbm.at[pl.ds(base, 8)], dst_v)
            pltpu.sync_copy(ew_hbm.at[pl.ds(base, 8)], ew_v)

            def blk(jj, c1):
                pltpu.async_copy(g_hbm.at[src_v.at[jj]], rows_v, gsem).wait()

                def ebody(e, c2):
                    w = plsc.load_gather(
                        ew_v,
                        [jnp.full((16,), jj, jnp.int32),
                         jnp.full((16,), e, jnp.int32)],
                    )
                    for k in range(HALF // 16):
                        msg_v[e, pl.ds(k * 16, 16)] = (
                            rows_v[e, pl.ds(k * 16, 16)] * w)
                    return c2

                lax.fori_loop(0, 128, ebody, 0)
                pltpu.sync_copy(msg_v, acc.at[dst_v.at[jj]], add=True)
                return c1

            lax.fori_loop(0, 8, blk, 0)
            return carry

        lax.fori_loop(0, RC // 8, gbody, 0)
        plsc.subcore_barrier()
        for i in range(NCHUNK):
            r0 = sid * RPT + i * 128
            pltpu.sync_copy(acc.at[pl.ds(r0, 128)], rows_v)
            pltpu.sync_copy(rows_v, out_hbm.at[pl.ds(r0, 128)])

    @pl.when(cid == 0)
    def _():
        run(g0_hbm, out0_hbm)

    @pl.when(cid == 1)
    def _():
        run(g1_hbm, out1_hbm)


def _mm1_body(x_ref, w_ref, parts_ref, g0_ref, g1_ref, dinv_ref):
    deg = jnp.sum(parts_ref[...], axis=0)
    dinv = jnp.where(deg > 0.0, lax.rsqrt(jnp.where(deg > 0.0, deg, 1.0)), 0.0)
    h = jnp.dot(x_ref[...], w_ref[...], preferred_element_type=jnp.float32)
    g = h * dinv[:, None]
    g0_ref[...] = g[:, :HALF]
    g1_ref[...] = g[:, HALF:]
    dinv_ref[...] = dinv


def _mm1(x_p, W1, parts):
    bm = 256
    grid = (NP // bm,)
    return pl.pallas_call(
        _mm1_body,
        grid=grid,
        in_specs=[
            pl.BlockSpec((bm, D_IN), lambda i: (i, 0)),
            pl.BlockSpec((D_IN, H), lambda i: (0, 0)),
            pl.BlockSpec((32, bm), lambda i: (0, i)),
        ],
        out_specs=[
            pl.BlockSpec((bm, HALF), lambda i: (i, 0)),
            pl.BlockSpec((bm, HALF), lambda i: (i, 0)),
            pl.BlockSpec((bm,), lambda i: (i,)),
        ],
        out_shape=[
            jax.ShapeDtypeStruct((NP, HALF), jnp.float32),
            jax.ShapeDtypeStruct((NP, HALF), jnp.float32),
            jax.ShapeDtypeStruct((NP,), jnp.float32),
        ],
    )(x_p, W1, parts)


def _mm2_body(a0_ref, a1_ref, dinv_ref, b_ref, w_ref, g0_ref, g1_ref):
    dinv = dinv_ref[...]
    acc = jnp.concatenate([a0_ref[...], a1_ref[...]], axis=1)
    a = jnp.maximum(acc * dinv[:, None] + b_ref[...][None, :], 0.0)
    h = jnp.dot(a, w_ref[...], preferred_element_type=jnp.float32)
    g = h * dinv[:, None]
    g0_ref[...] = g[:, :HALF]
    g1_ref[...] = g[:, HALF:]


def _mm2(acc0, acc1, dinv, b1, W2):
    bm = 256
    grid = (NP // bm,)
    return pl.pallas_call(
        _mm2_body,
        grid=grid,
        in_specs=[
            pl.BlockSpec((bm, HALF), lambda i: (i, 0)),
            pl.BlockSpec((bm, HALF), lambda i: (i, 0)),
            pl.BlockSpec((bm,), lambda i: (i,)),
            pl.BlockSpec((H,), lambda i: (0,)),
            pl.BlockSpec((H, H), lambda i: (0, 0)),
        ],
        out_specs=[
            pl.BlockSpec((bm, HALF), lambda i: (i, 0)),
            pl.BlockSpec((bm, HALF), lambda i: (i, 0)),
        ],
        out_shape=[
            jax.ShapeDtypeStruct((NP, HALF), jnp.float32),
            jax.ShapeDtypeStruct((NP, HALF), jnp.float32),
        ],
    )(acc0, acc1, dinv, b1, W2)


def _fin_body(a0_ref, a1_ref, dinv_ref, b_ref, o_ref):
    dinv = dinv_ref[...]
    acc = jnp.concatenate([a0_ref[...], a1_ref[...]], axis=1)
    o_ref[...] = jnp.maximum(acc * dinv[:, None] + b_ref[...][None, :], 0.0)


def _fin(acc0, acc1, dinv, b2):
    bm = 256
    grid = (NP // bm,)
    return pl.pallas_call(
        _fin_body,
        grid=grid,
        in_specs=[
            pl.BlockSpec((bm, HALF), lambda i: (i, 0)),
            pl.BlockSpec((bm, HALF), lambda i: (i, 0)),
            pl.BlockSpec((bm,), lambda i: (i,)),
            pl.BlockSpec((H,), lambda i: (0,)),
        ],
        out_specs=pl.BlockSpec((bm, H), lambda i: (i, 0)),
        out_shape=jax.ShapeDtypeStruct((NP, H), jnp.float32),
    )(acc0, acc1, dinv, b2)


def kernel(x, edge_index, edge_weight, W1, b1, W2, b2):
    src = edge_index[0].astype(jnp.int32)
    dst = edge_index[1].astype(jnp.int32)
    loop = jnp.arange(N, dtype=jnp.int32)
    padi = jnp.zeros((E_PAD - E_FULL,), jnp.int32)
    padf = jnp.zeros((E_PAD - E_FULL,), jnp.float32)
    src2 = jnp.concatenate([src, loop, padi]).reshape(ROWS, 128)
    dst2 = jnp.concatenate([dst, loop, padi]).reshape(ROWS, 128)
    ew1 = jnp.concatenate(
        [edge_weight.astype(jnp.float32), jnp.ones((N,), jnp.float32), padf]
    )
    ew2 = ew1.reshape(ROWS, 128)
    x_p = jnp.pad(x, ((0, NP - N), (0, 0)))

    parts = _deg_kernel(dst2.reshape(-1), ew1).reshape(32, NP)
    g0, g1, dinv = _mm1(x_p, W1, parts)
    a0, a1 = _agg_kernel(src2, dst2, ew2, g0, g1)
    g20, g21 = _mm2(a0, a1, dinv, b1, W2)
    c0, c1 = _agg_kernel(src2, dst2, ew2, g20, g21)
    out = _fin(c0, c1, dinv, b2)
    return out[:N]
